# initial kernel scaffold (unmeasured)
import jax
import jax.numpy as jnp
from jax import lax
from jax.experimental import pallas as pl
from jax.experimental.pallas import tpu as pltpu

N_ROWS = 4096
N_COLS = 4096
BLK = 512
N_BLK = N_ROWS // BLK


def kernel(partial, gamma):
    partial2d = partial.reshape(2 * N_ROWS, N_COLS)
    gamma2d = gamma.reshape(1, N_COLS)

    def body(partial_ref, gamma_ref, out_ref, remote_ref,
             mine_v, rem_v, out_v, local_sems, send_sems, recv_sems):
        x = lax.axis_index("x")
        y = lax.axis_index("y")
        z = lax.axis_index("z")
        my_base = y * N_ROWS
        nbr_base = (1 - y) * N_ROWS

        sends = []
        for b in range(N_BLK):
            rdma = pltpu.make_async_remote_copy(
                src_ref=partial_ref.at[pl.ds(nbr_base + b * BLK, BLK), :],
                dst_ref=remote_ref.at[pl.ds(b * BLK, BLK), :],
                send_sem=send_sems.at[b],
                recv_sem=recv_sems.at[b],
                device_id=(x, 1 - y, z),
                device_id_type=pltpu.DeviceIdType.MESH,
            )
            rdma.start()
            sends.append(rdma)

        for b in range(N_BLK):
            recv = pltpu.make_async_remote_copy(
                src_ref=partial_ref.at[pl.ds(nbr_base + b * BLK, BLK), :],
                dst_ref=remote_ref.at[pl.ds(b * BLK, BLK), :],
                send_sem=send_sems.at[b],
                recv_sem=recv_sems.at[b],
                device_id=(x, 1 - y, z),
                device_id_type=pltpu.DeviceIdType.MESH,
            )
            recv.wait_recv()

            cp_mine = pltpu.make_async_copy(
                partial_ref.at[pl.ds(my_base + b * BLK, BLK), :],
                mine_v, local_sems.at[0])
            cp_mine.start()
            cp_rem = pltpu.make_async_copy(
                remote_ref.at[pl.ds(b * BLK, BLK), :],
                rem_v, local_sems.at[1])
            cp_rem.start()
            cp_mine.wait()
            cp_rem.wait()

            s = mine_v[...] + rem_v[...]
            ss = jnp.sum(s * s, axis=1, keepdims=True)
            rms = jnp.sqrt(ss / N_COLS + 1e-6)
            out_v[...] = s / rms * gamma_ref[...]

            cp_out = pltpu.make_async_copy(
                out_v, out_ref.at[pl.ds(b * BLK, BLK), :], local_sems.at[2])
            cp_out.start()
            cp_out.wait()

        for rdma in sends:
            rdma.wait_send()

    out, _ = pl.pallas_call(
        body,
        out_shape=(
            jax.ShapeDtypeStruct((N_ROWS, N_COLS), jnp.float32),
            jax.ShapeDtypeStruct((N_ROWS, N_COLS), jnp.float32),
        ),
        in_specs=[
            pl.BlockSpec(memory_space=pltpu.ANY),
            pl.BlockSpec(memory_space=pltpu.VMEM),
        ],
        out_specs=(
            pl.BlockSpec(memory_space=pltpu.ANY),
            pl.BlockSpec(memory_space=pltpu.ANY),
        ),
        scratch_shapes=[
            pltpu.VMEM((BLK, N_COLS), jnp.float32),
            pltpu.VMEM((BLK, N_COLS), jnp.float32),
            pltpu.VMEM((BLK, N_COLS), jnp.float32),
            pltpu.SemaphoreType.DMA((3,)),
            pltpu.SemaphoreType.DMA((N_BLK,)),
            pltpu.SemaphoreType.DMA((N_BLK,)),
        ],
        compiler_params=pltpu.CompilerParams(collective_id=0),
    )(partial2d, gamma2d)
    return out


# baseline (device time: 780146 ns/iter reference)
import jax
import jax.numpy as jnp
from jax import lax
from jax.experimental import pallas as pl
from jax.experimental.pallas import tpu as pltpu

N_ROWS = 4096
N_COLS = 4096
BLK = 256
N_BLK = N_ROWS // BLK


def kernel(partial, gamma):
    partial2d = partial.reshape(2 * N_ROWS, N_COLS)
    gamma2d = gamma.reshape(1, N_COLS)

    def body(partial_ref, gamma_ref, out_ref, remote_ref,
             mine_v, rem_v, out_v, local_sems, send_sems, recv_sems):
        x = lax.axis_index("x")
        y = lax.axis_index("y")
        z = lax.axis_index("z")
        my_base = y * N_ROWS
        nbr_base = (1 - y) * N_ROWS

        sends = []
        for b in range(N_BLK):
            rdma = pltpu.make_async_remote_copy(
                src_ref=partial_ref.at[pl.ds(nbr_base + b * BLK, BLK), :],
                dst_ref=remote_ref.at[pl.ds(b * BLK, BLK), :],
                send_sem=send_sems.at[b],
                recv_sem=recv_sems.at[b],
                device_id=(x, 1 - y, z),
            )
            rdma.start()
            sends.append(rdma)

        for b in range(N_BLK):
            recv = pltpu.make_async_remote_copy(
                src_ref=partial_ref.at[pl.ds(nbr_base + b * BLK, BLK), :],
                dst_ref=remote_ref.at[pl.ds(b * BLK, BLK), :],
                send_sem=send_sems.at[b],
                recv_sem=recv_sems.at[b],
                device_id=(x, 1 - y, z),
            )
            recv.wait_recv()

            cp_mine = pltpu.make_async_copy(
                partial_ref.at[pl.ds(my_base + b * BLK, BLK), :],
                mine_v, local_sems.at[0])
            cp_mine.start()
            cp_rem = pltpu.make_async_copy(
                remote_ref.at[pl.ds(b * BLK, BLK), :],
                rem_v, local_sems.at[1])
            cp_rem.start()
            cp_mine.wait()
            cp_rem.wait()

            s = mine_v[...] + rem_v[...]
            ss = jnp.sum(s * s, axis=1, keepdims=True)
            rms = jnp.sqrt(ss / N_COLS + 1e-6)
            out_v[...] = s / rms * gamma_ref[...]

            cp_out = pltpu.make_async_copy(
                out_v, out_ref.at[pl.ds(b * BLK, BLK), :], local_sems.at[2])
            cp_out.start()
            cp_out.wait()

        for rdma in sends:
            rdma.wait_send()

    out, _ = pl.pallas_call(
        body,
        out_shape=(
            jax.ShapeDtypeStruct((N_ROWS, N_COLS), jnp.float32),
            jax.ShapeDtypeStruct((N_ROWS, N_COLS), jnp.float32),
        ),
        in_specs=[
            pl.BlockSpec(memory_space=pl.ANY),
            pl.BlockSpec(memory_space=pltpu.VMEM),
        ],
        out_specs=(
            pl.BlockSpec(memory_space=pl.ANY),
            pl.BlockSpec(memory_space=pl.ANY),
        ),
        scratch_shapes=[
            pltpu.VMEM((BLK, N_COLS), jnp.float32),
            pltpu.VMEM((BLK, N_COLS), jnp.float32),
            pltpu.VMEM((BLK, N_COLS), jnp.float32),
            pltpu.SemaphoreType.DMA((3,)),
            pltpu.SemaphoreType.DMA((N_BLK,)),
            pltpu.SemaphoreType.DMA((N_BLK,)),
        ],
    )(partial2d, gamma2d)
    return out


# device time: 419529 ns/iter; 1.8596x vs baseline; 1.8596x over previous
import jax
import jax.numpy as jnp
from jax import lax
from jax.experimental import pallas as pl
from jax.experimental.pallas import tpu as pltpu

N_ROWS = 4096
N_COLS = 4096
QROWS = 1024
BLK = 256
JBLK = QROWS // BLK
N_SEM = 4 * JBLK


def kernel(partial, gamma):
    partial2d = partial.reshape(2 * N_ROWS, N_COLS)
    gamma2d = gamma.reshape(1, N_COLS)

    def body(partial_ref, gamma_ref, out_ref, remote_ref,
             mine_v, rem_v, out_v, local_sems, send_sems, recv_sems):
        x = lax.axis_index("x")
        y = lax.axis_index("y")
        z = lax.axis_index("z")
        my_base = y * N_ROWS
        nbr_base = (1 - y) * N_ROWS
        g = 2 * x + z
        q_own = QROWS * g
        q_diag = QROWS * (3 - g)
        q_x = QROWS * (2 * (1 - x) + z)
        q_z = QROWS * (2 * x + (1 - z))

        sends = []

        def y_send(qbase, j, slot):
            rows = qbase + j * BLK
            rdma = pltpu.make_async_remote_copy(
                src_ref=partial_ref.at[pl.ds(nbr_base + rows, BLK), :],
                dst_ref=remote_ref.at[pl.ds(rows, BLK), :],
                send_sem=send_sems.at[slot],
                recv_sem=recv_sems.at[slot],
                device_id=(x, 1 - y, z),
            )
            rdma.start()
            sends.append(rdma)

        def recv_wait(qbase, j, slot):
            rows = qbase + j * BLK
            recv = pltpu.make_async_remote_copy(
                src_ref=partial_ref.at[pl.ds(nbr_base + rows, BLK), :],
                dst_ref=remote_ref.at[pl.ds(rows, BLK), :],
                send_sem=send_sems.at[slot],
                recv_sem=recv_sems.at[slot],
                device_id=(x, 1 - y, z),
            )
            recv.wait_recv()

        def forward(j):
            rows = q_own + j * BLK
            for slot, dev in ((8 + j, (1 - x, y, z)), (12 + j, (x, y, 1 - z))):
                rdma = pltpu.make_async_remote_copy(
                    src_ref=remote_ref.at[pl.ds(rows, BLK), :],
                    dst_ref=remote_ref.at[pl.ds(rows, BLK), :],
                    send_sem=send_sems.at[slot],
                    recv_sem=recv_sems.at[slot],
                    device_id=dev,
                )
                rdma.start()
                sends.append(rdma)

        def compute(qbase, j):
            rows = qbase + j * BLK
            cp_mine = pltpu.make_async_copy(
                partial_ref.at[pl.ds(my_base + rows, BLK), :],
                mine_v, local_sems.at[0])
            cp_mine.start()
            cp_rem = pltpu.make_async_copy(
                remote_ref.at[pl.ds(rows, BLK), :],
                rem_v, local_sems.at[1])
            cp_rem.start()
            cp_mine.wait()
            cp_rem.wait()

            s = mine_v[...] + rem_v[...]
            ss = jnp.sum(s * s, axis=1, keepdims=True)
            rms = jnp.sqrt(ss / N_COLS + 1e-6)
            out_v[...] = s / rms * gamma_ref[...]

            cp_out = pltpu.make_async_copy(
                out_v, out_ref.at[pl.ds(rows, BLK), :], local_sems.at[2])
            cp_out.start()
            cp_out.wait()

        for j in range(JBLK):
            y_send(q_own, j, j)
        for j in range(JBLK):
            y_send(q_diag, j, 4 + j)

        for j in range(JBLK):
            recv_wait(q_own, j, j)
            forward(j)
            compute(q_own, j)

        order = [(q_x, 0, 8), (q_z, 0, 12),
                 (q_x, 1, 9), (q_z, 1, 13),
                 (q_x, 2, 10), (q_z, 2, 14),
                 (q_diag, 0, 4),
                 (q_x, 3, 11), (q_z, 3, 15),
                 (q_diag, 1, 5), (q_diag, 2, 6), (q_diag, 3, 7)]
        for qbase, j, slot in order:
            recv_wait(qbase, j, slot)
            compute(qbase, j)

        for rdma in sends:
            rdma.wait_send()

    out, _ = pl.pallas_call(
        body,
        out_shape=(
            jax.ShapeDtypeStruct((N_ROWS, N_COLS), jnp.float32),
            jax.ShapeDtypeStruct((N_ROWS, N_COLS), jnp.float32),
        ),
        in_specs=[
            pl.BlockSpec(memory_space=pl.ANY),
            pl.BlockSpec(memory_space=pltpu.VMEM),
        ],
        out_specs=(
            pl.BlockSpec(memory_space=pl.ANY),
            pl.BlockSpec(memory_space=pl.ANY),
        ),
        scratch_shapes=[
            pltpu.VMEM((BLK, N_COLS), jnp.float32),
            pltpu.VMEM((BLK, N_COLS), jnp.float32),
            pltpu.VMEM((BLK, N_COLS), jnp.float32),
            pltpu.SemaphoreType.DMA((3,)),
            pltpu.SemaphoreType.DMA((N_SEM,)),
            pltpu.SemaphoreType.DMA((N_SEM,)),
        ],
    )(partial2d, gamma2d)
    return out
